# baseline (speedup 1.0000x reference)
import jax
import jax.numpy as jnp
from jax import lax
from jax.experimental import pallas as pl
from jax.experimental.pallas import tpu as pltpu
from jax.experimental.pallas import tpu_sc as plsc

D = 128
N0, N1, N2 = 50000, 10000, 2000
ETOT = 800000
E2 = 20000
NC, NS, L = 2, 16, 16
NW = NC * NS
CH = 128
K = 5
EPW = CH * K
E2P = NW * EPW
BE = 512
NBLK = E2P // BE
F32 = jnp.float32

_MESH = plsc.VectorSubcoreMesh(core_axis_name="c", subcore_axis_name="s")


C0 = 5
C1 = 5
NCHK = NW * K
CM = max(C0, C1)


def _sc1_body(t1t_hbm, t2t_hbm, uidxt_hbm, eid_hbm, src_hbm, dst_hbm,
              x_hbm, emb_hbm,
              t1g_hbm, t2g_hbm, xj_hbm, xi_hbm, user_hbm,
              eid_v, src_v, dst_v, t1_b, t2_b, uidx_b,
              xj_b, xi_b, user_b, sems):
    c = lax.axis_index("c")
    s = lax.axis_index("s")

    def run(chunk0, n):
        pltpu.sync_copy(eid_hbm.at[pl.ds(chunk0 * CH, n * CH)],
                        eid_v.at[pl.ds(0, n * CH)])
        pltpu.sync_copy(src_hbm.at[pl.ds(chunk0 * CH, n * CH)],
                        src_v.at[pl.ds(0, n * CH)])
        pltpu.sync_copy(dst_hbm.at[pl.ds(chunk0 * CH, n * CH)],
                        dst_v.at[pl.ds(0, n * CH)])
        descs = {}

        def fire_indep(j):
            par = j % 2
            descs[(j, 0)] = pltpu.async_copy(
                t1t_hbm.at[eid_v.at[pl.ds(j * CH, CH)]], t1_b.at[par], sems.at[par, 0])
            descs[(j, 1)] = pltpu.async_copy(
                t2t_hbm.at[eid_v.at[pl.ds(j * CH, CH)]], t2_b.at[par], sems.at[par, 1])
            descs[(j, 2)] = pltpu.async_copy(
                uidxt_hbm.at[eid_v.at[pl.ds(j * CH, CH)]], uidx_b.at[par], sems.at[par, 2])
            descs[(j, 3)] = pltpu.async_copy(
                x_hbm.at[src_v.at[pl.ds(j * CH, CH)]], xj_b.at[par], sems.at[par, 3])
            descs[(j, 4)] = pltpu.async_copy(
                x_hbm.at[dst_v.at[pl.ds(j * CH, CH)]], xi_b.at[par], sems.at[par, 4])

        def fire_emb(j):
            par = j % 2
            descs[(j, 2)].wait()
            descs[(j, 5)] = pltpu.async_copy(
                emb_hbm.at[uidx_b.at[par]], user_b.at[par], sems.at[par, 5])

        def drain_store(j):
            par = j % 2
            for t in (0, 1, 3, 4, 5):
                descs[(j, t)].wait()
            base = (chunk0 + j) * CH
            pltpu.sync_copy(t1_b.at[par], t1g_hbm.at[pl.ds(base, CH)])
            pltpu.sync_copy(t2_b.at[par], t2g_hbm.at[pl.ds(base, CH)])
            pltpu.sync_copy(xj_b.at[par], xj_hbm.at[pl.ds(base, CH)])
            pltpu.sync_copy(xi_b.at[par], xi_hbm.at[pl.ds(base, CH)])
            pltpu.sync_copy(user_b.at[par], user_hbm.at[pl.ds(base, CH)])

        fire_indep(0)
        fire_emb(0)
        if n > 1:
            fire_indep(1)
        for j in range(n):
            drain_store(j)
            if j + 1 < n:
                fire_emb(j + 1)
            if j + 2 < n:
                fire_indep(j + 2)

    @pl.when(c == 0)
    def _():
        run(s * C0, C0)

    @pl.when(c == 1)
    def _():
        run(NS * C0 + s * C1, C1)


_sc1 = pl.kernel(
    _sc1_body,
    out_type=[
        jax.ShapeDtypeStruct((E2P,), F32),
        jax.ShapeDtypeStruct((E2P,), F32),
        jax.ShapeDtypeStruct((E2P, D), F32),
        jax.ShapeDtypeStruct((E2P, D), F32),
        jax.ShapeDtypeStruct((E2P, D), F32),
    ],
    mesh=_MESH,
    scratch_types=[
        pltpu.VMEM((CM * CH,), jnp.int32),
        pltpu.VMEM((CM * CH,), jnp.int32),
        pltpu.VMEM((CM * CH,), jnp.int32),
        pltpu.VMEM((2, CH), F32),
        pltpu.VMEM((2, CH), F32),
        pltpu.VMEM((2, CH), jnp.int32),
        pltpu.VMEM((2, CH, D), F32),
        pltpu.VMEM((2, CH, D), F32),
        pltpu.VMEM((2, CH, D), F32),
        pltpu.SemaphoreType.DMA((2, 6)),
    ],
)


_CC = (0.9999982503105564, -19.738913224823705, 64.92748557653424,
       -85.26424585397747, 58.77468699833364, -21.06805280070973)


def _fcos(y):
    fr = y - jnp.round(y)
    v = fr * fr
    acc = jnp.full_like(v, _CC[5])
    for k in (4, 3, 2, 1, 0):
        acc = acc * v + _CC[k]
    return acc


def _tc2_body(t1_ref, t2_ref, dst_ref, xj_ref, xi_ref, user_ref,
              freqc_ref, tbc_ref, linw1_ref, linw2_ref, linb_ref,
              wv1_ref, wv2_ref, wv3_ref, ffnw_ref, ffnb_ref,
              lng_ref, lnb_ref, m1b_ref, agg_ref,
              w1_s, wta_s, wtb_s, w3_s, brow_s, itr_s, g_s, browm_s):
    i = pl.program_id(0)
    dot0 = lambda a, b: jax.lax.dot_general(
        a, b, (((0,), (0,)), ((), ())), preferred_element_type=F32)
    dot = lambda a, b: jax.lax.dot_general(
        a, b, (((1,), (0,)), ((), ())), preferred_element_type=F32)
    freqc = freqc_ref[...]
    tbc = tbc_ref[...]

    @pl.when(i == 0)
    def _():
        f = ffnw_ref[...]
        wv2f = dot(wv2_ref[...], f)
        w1_s[...] = dot(wv1_ref[...], f)
        wta_s[...] = dot(linw1_ref[...], wv2f)
        wtb_s[...] = dot(linw2_ref[...], wv2f)
        w3_s[...] = dot(wv3_ref[...], f)
        brow_s[...] = dot(linb_ref[...], wv2f) + ffnb_ref[...]
        agg_ref[...] = jnp.zeros_like(agg_ref)
        itc = _fcos(tbc)
        itr_s[...] = (dot0(itc, linw1_ref[...]) + dot0(itc, linw2_ref[...])
                      + linb_ref[...])
        g_s[...] = m1b_ref[...] * lng_ref[...].reshape(3 * D, 1)
        browm_s[...] = dot(lnb_ref[...], m1b_ref[...])

    t1 = t1_ref[0]
    t2 = t2_ref[0]
    t1e_t = _fcos(freqc * t1 + tbc)
    t2e_t = _fcos(freqc * t2 + tbc)
    itr = itr_s[...]
    xj = xj_ref[...]
    xi = xi_ref[...]
    user = user_ref[...]
    out = (dot(xj, w1_s[...]) + dot0(t1e_t, wta_s[...])
           + dot0(t2e_t, wtb_s[...]) + dot(user, w3_s[...]) + brow_s[...])
    q = jnp.concatenate([xi, jnp.broadcast_to(itr, xi.shape), user], axis=1)
    z = out + q
    o384 = jnp.ones((3 * D, 8), F32)
    m = dot(z, o384)[:, 0:1] * F32(1.0 / (3 * D))
    zc = z - m
    var = dot(zc * zc, o384)[:, 0:1] * F32(1.0 / (3 * D))
    r = lax.rsqrt(var + 1e-5)
    msgp = r * dot(zc, g_s[...]) + browm_s[...]
    seg = lax.broadcasted_iota(jnp.int32, (N2, BE), 0)
    oh = (seg == dst_ref[0]).astype(F32)
    agg_ref[...] += dot(oh, msgp)


def _tc2(t13, t23, dst3, xj, xi, user, freqc, tbc, linw1, linw2, linb,
         wv1, wv2, wv3, ffnw, ffnb, lng, lnb, m1b):
    full = lambda shape: pl.BlockSpec(shape, lambda i: tuple(0 for _ in shape))
    return pl.pallas_call(
        _tc2_body,
        grid=(NBLK,),
        in_specs=[
            pl.BlockSpec((1, 1, BE), lambda i: (i, 0, 0)),
            pl.BlockSpec((1, 1, BE), lambda i: (i, 0, 0)),
            pl.BlockSpec((1, 1, BE), lambda i: (i, 0, 0)),
            pl.BlockSpec((BE, D), lambda i: (i, 0)),
            pl.BlockSpec((BE, D), lambda i: (i, 0)),
            pl.BlockSpec((BE, D), lambda i: (i, 0)),
            full((D, 1)), full((D, 1)),
            full((D, D)), full((D, D)), full((1, D)),
            full((D, 3 * D)), full((D, 3 * D)), full((D, 3 * D)),
            full((3 * D, 3 * D)), full((1, 3 * D)),
            full((1, 3 * D)), full((1, 3 * D)), full((3 * D, D)),
        ],
        out_specs=pl.BlockSpec((N2, D), lambda i: (0, 0)),
        out_shape=jax.ShapeDtypeStruct((N2, D), F32),
        scratch_shapes=[
            pltpu.VMEM((D, 3 * D), F32),
            pltpu.VMEM((D, 3 * D), F32),
            pltpu.VMEM((D, 3 * D), F32),
            pltpu.VMEM((D, 3 * D), F32),
            pltpu.VMEM((1, 3 * D), F32),
            pltpu.VMEM((1, D), F32),
            pltpu.VMEM((3 * D, D), F32),
            pltpu.VMEM((1, D), F32),
        ],
    )(t13, t23, dst3, xj, xi, user, freqc, tbc, linw1, linw2, linb,
      wv1, wv2, wv3, ffnw, ffnb, lng, lnb, m1b)


def _tc4_body(x_ref, agg_ref, m1a_ref, m1bias_ref, m2_ref, m2b_ref, h_ref):
    i = pl.program_id(0)
    dot = lambda a, b: jax.lax.dot_general(
        a, b, (((1,), (0,)), ((), ())), preferred_element_type=F32)
    base = dot(x_ref[...], m1a_ref[...]) + m1bias_ref[...]

    def finish(acc):
        acc = jnp.where(acc > 0, acc, 0.1 * acc)
        h_ref[...] = dot(acc, m2_ref[...]) + m2b_ref[...]

    @pl.when(i < N2 // 400)
    def _():
        finish(base + agg_ref[...])

    @pl.when(i >= N2 // 400)
    def _():
        finish(base)


def _tc4(x, agg, m1a, m1bias, m2, m2b):
    BR = 400
    full = lambda shape: pl.BlockSpec(shape, lambda i: tuple(0 for _ in shape))
    return pl.pallas_call(
        _tc4_body,
        grid=(N1 // BR,),
        in_specs=[
            pl.BlockSpec((BR, D), lambda i: (i, 0)),
            pl.BlockSpec((BR, D), lambda i: (jnp.minimum(i, N2 // BR - 1), 0)),
            full((D, D)), full((1, D)),
            full((D, D)), full((1, D)),
        ],
        out_specs=pl.BlockSpec((BR, D), lambda i: (i, 0)),
        out_shape=jax.ShapeDtypeStruct((N1, D), F32),
    )(x, agg, m1a, m1bias, m2, m2b)


def _sc5_body(h_hbm, src_hbm, hs_hbm, src_v, h_b, sems):
    c = lax.axis_index("c")
    s = lax.axis_index("s")

    def run(chunk0, n):
        pltpu.sync_copy(src_hbm.at[pl.ds(chunk0 * CH, n * CH)],
                        src_v.at[pl.ds(0, n * CH)])
        descs = {}

        def fire(j):
            par = j % 2
            descs[j] = pltpu.async_copy(
                h_hbm.at[src_v.at[pl.ds(j * CH, CH)]], h_b.at[par], sems.at[par])

        fire(0)
        if n > 1:
            fire(1)
        for j in range(n):
            descs[j].wait()
            pltpu.sync_copy(h_b.at[j % 2],
                            hs_hbm.at[pl.ds((chunk0 + j) * CH, CH)])
            if j + 2 < n:
                fire(j + 2)

    @pl.when(c == 0)
    def _():
        run(s * C0, C0)

    @pl.when(c == 1)
    def _():
        run(NS * C0 + s * C1, C1)


_sc5 = pl.kernel(
    _sc5_body,
    out_type=jax.ShapeDtypeStruct((E2P, D), F32),
    mesh=_MESH,
    scratch_types=[
        pltpu.VMEM((CM * CH,), jnp.int32),
        pltpu.VMEM((2, CH, D), F32),
        pltpu.SemaphoreType.DMA((2,)),
    ],
)


def _tc6_body(hs_ref, dst_ref, h_ref, lw_ref, lb_ref, rw_ref, out_ref,
              sacc, cacc):
    i = pl.program_id(0)
    dot = lambda a, b: jax.lax.dot_general(
        a, b, (((1,), (0,)), ((), ())), preferred_element_type=F32)
    seg = lax.broadcasted_iota(jnp.int32, (N2, BE), 0)
    oh = (seg == dst_ref[0]).astype(F32)

    @pl.when(i == 0)
    def _():
        sacc[...] = jnp.zeros_like(sacc)
        cacc[...] = jnp.zeros_like(cacc)

    sacc[...] += dot(oh, hs_ref[...])
    cacc[...] += dot(oh, jnp.ones((BE, 8), F32))

    @pl.when(i == NBLK - 1)
    def _():
        cnt = cacc[:, 0:1]
        mean = sacc[...] / jnp.maximum(cnt, 1.0)
        out = dot(mean, lw_ref[...]) + lb_ref[...] + dot(h_ref[...], rw_ref[...])
        nrm = jnp.sqrt(jnp.sum(out * out, axis=-1, keepdims=True))
        out_ref[...] = out / jnp.maximum(nrm, 1e-12)


def _tc6(hs, dst3, h, lw, lb, rw):
    full = lambda shape: pl.BlockSpec(shape, lambda i: tuple(0 for _ in shape))
    return pl.pallas_call(
        _tc6_body,
        grid=(NBLK,),
        in_specs=[
            pl.BlockSpec((BE, D), lambda i: (i, 0)),
            pl.BlockSpec((1, 1, BE), lambda i: (i, 0, 0)),
            pl.BlockSpec((N2, D), lambda i: (0, 0)),
            full((D, D)), full((1, D)), full((D, D)),
        ],
        out_specs=pl.BlockSpec((N2, D), lambda i: (0, 0)),
        out_shape=jax.ShapeDtypeStruct((N2, D), F32),
        scratch_shapes=[
            pltpu.VMEM((N2, D), F32),
            pltpu.VMEM((N2, 8), F32),
        ],
    )(hs, dst3, h, lw, lb, rw)


def kernel(x, edge_index1, e_id1, edge_index2, e_id2, emb, tg_edge_attr,
           params, size1, size2):
    p = params
    uidx_tab = tg_edge_attr[:, 2].astype(jnp.int32)
    t1_tab = tg_edge_attr[:, 0]
    t2_tab = tg_edge_attr[:, 1]
    pad = E2P - E2
    eid3 = jnp.pad(e_id2.astype(jnp.int32), (0, pad))
    src3 = jnp.pad(edge_index2[0].astype(jnp.int32), (0, pad))
    dst_pad = jnp.pad(edge_index2[1].astype(jnp.int32), (0, pad),
                      constant_values=N2)
    dst3 = dst_pad
    dstl = dst_pad.reshape(NBLK, 1, BE)

    row = lambda v: v.reshape(1, -1)
    col = lambda v: v.reshape(-1, 1)
    t1g, t2g, xj, xi, user = _sc1(t1_tab, t2_tab, uidx_tab,
                                  eid3, src3, dst3, x, emb)
    t13 = t1g.reshape(NBLK, 1, BE)
    t23 = t2g.reshape(NBLK, 1, BE)
    inv2pi = F32(1.0 / (2.0 * jnp.pi))
    agg = _tc2(t13, t23, dstl, xj, xi, user,
               col(p['basic_freq']) * inv2pi, col(p['t_bias']) * inv2pi,
               p['lin_w'][:D], p['lin_w'][D:], row(p['lin_b']),
               p['wv'][:D], p['wv'][D:2 * D], p['wv'][2 * D:],
               p['ffn_w'], row(p['ffn_b']),
               row(p['ln_g']), row(p['ln_b']), p['m1_w'][D:])
    h = _tc4(x, agg, p['m1_w'][:D], row(p['m1_b']),
             p['m2_w'], row(p['m2_b']))
    hs = _sc5(h, src3)
    return _tc6(hs, dstl, h, p['sage1_lw'], row(p['sage1_lb']), p['sage1_rw'])


# speedup vs baseline: 1.0343x; 1.0343x over previous
import jax
import jax.numpy as jnp
from jax import lax
from jax.experimental import pallas as pl
from jax.experimental.pallas import tpu as pltpu
from jax.experimental.pallas import tpu_sc as plsc

D = 128
N0, N1, N2 = 50000, 10000, 2000
ETOT = 800000
E2 = 20000
NC, NS, L = 2, 16, 16
NW = NC * NS
CH = 128
K = 5
EPW = CH * K
E2P = NW * EPW
BE = 512
NBLK = E2P // BE
F32 = jnp.float32

_MESH = plsc.VectorSubcoreMesh(core_axis_name="c", subcore_axis_name="s")


C0 = 5
C1 = 5
NCHK = NW * K
CM = max(C0, C1)


def _sc1_body(t1t_hbm, t2t_hbm, uidxt_hbm, eid_hbm, src_hbm, dst_hbm,
              x_hbm, emb_hbm,
              t1g_hbm, t2g_hbm, xj_hbm, xi_hbm, user_hbm,
              eid_v, src_v, dst_v, t1_b, t2_b, uidx_b,
              xj_b, xi_b, user_b, sems):
    c = lax.axis_index("c")
    s = lax.axis_index("s")

    def run(chunk0, n):
        pltpu.sync_copy(eid_hbm.at[pl.ds(chunk0 * CH, n * CH)],
                        eid_v.at[pl.ds(0, n * CH)])
        pltpu.sync_copy(src_hbm.at[pl.ds(chunk0 * CH, n * CH)],
                        src_v.at[pl.ds(0, n * CH)])
        pltpu.sync_copy(dst_hbm.at[pl.ds(chunk0 * CH, n * CH)],
                        dst_v.at[pl.ds(0, n * CH)])
        descs = {}

        def fire_indep(j):
            par = j % 2
            descs[(j, 0)] = pltpu.async_copy(
                t1t_hbm.at[eid_v.at[pl.ds(j * CH, CH)]], t1_b.at[par], sems.at[par, 0])
            descs[(j, 1)] = pltpu.async_copy(
                t2t_hbm.at[eid_v.at[pl.ds(j * CH, CH)]], t2_b.at[par], sems.at[par, 1])
            descs[(j, 2)] = pltpu.async_copy(
                uidxt_hbm.at[eid_v.at[pl.ds(j * CH, CH)]], uidx_b.at[par], sems.at[par, 2])
            descs[(j, 3)] = pltpu.async_copy(
                x_hbm.at[src_v.at[pl.ds(j * CH, CH)]], xj_b.at[par], sems.at[par, 3])
            descs[(j, 4)] = pltpu.async_copy(
                x_hbm.at[dst_v.at[pl.ds(j * CH, CH)]], xi_b.at[par], sems.at[par, 4])

        def fire_emb(j):
            par = j % 2
            descs[(j, 2)].wait()
            descs[(j, 5)] = pltpu.async_copy(
                emb_hbm.at[uidx_b.at[par]], user_b.at[par], sems.at[par, 5])

        def drain_store(j):
            par = j % 2
            for t in (0, 1, 3, 4, 5):
                descs[(j, t)].wait()
            base = (chunk0 + j) * CH
            pltpu.sync_copy(t1_b.at[par], t1g_hbm.at[pl.ds(base, CH)])
            pltpu.sync_copy(t2_b.at[par], t2g_hbm.at[pl.ds(base, CH)])
            pltpu.sync_copy(xj_b.at[par], xj_hbm.at[pl.ds(base, CH)])
            pltpu.sync_copy(xi_b.at[par], xi_hbm.at[pl.ds(base, CH)])
            pltpu.sync_copy(user_b.at[par], user_hbm.at[pl.ds(base, CH)])

        fire_indep(0)
        fire_emb(0)
        if n > 1:
            fire_indep(1)
        for j in range(n):
            drain_store(j)
            if j + 1 < n:
                fire_emb(j + 1)
            if j + 2 < n:
                fire_indep(j + 2)

    @pl.when(c == 0)
    def _():
        run(s * C0, C0)

    @pl.when(c == 1)
    def _():
        run(NS * C0 + s * C1, C1)


_sc1 = pl.kernel(
    _sc1_body,
    out_type=[
        jax.ShapeDtypeStruct((E2P,), F32),
        jax.ShapeDtypeStruct((E2P,), F32),
        jax.ShapeDtypeStruct((E2P, D), F32),
        jax.ShapeDtypeStruct((E2P, D), F32),
        jax.ShapeDtypeStruct((E2P, D), F32),
    ],
    mesh=_MESH,
    scratch_types=[
        pltpu.VMEM((CM * CH,), jnp.int32),
        pltpu.VMEM((CM * CH,), jnp.int32),
        pltpu.VMEM((CM * CH,), jnp.int32),
        pltpu.VMEM((2, CH), F32),
        pltpu.VMEM((2, CH), F32),
        pltpu.VMEM((2, CH), jnp.int32),
        pltpu.VMEM((2, CH, D), F32),
        pltpu.VMEM((2, CH, D), F32),
        pltpu.VMEM((2, CH, D), F32),
        pltpu.SemaphoreType.DMA((2, 6)),
    ],
)


_CC = (0.9999982503105564, -19.738913224823705, 64.92748557653424,
       -85.26424585397747, 58.77468699833364, -21.06805280070973)


def _fcos(y):
    fr = y - jnp.round(y)
    v = fr * fr
    acc = jnp.full_like(v, _CC[5])
    for k in (4, 3, 2, 1, 0):
        acc = acc * v + _CC[k]
    return acc


def _tc2_body(t1_ref, t2_ref, dst_ref, xj_ref, xi_ref, user_ref,
              freqc_ref, tbc_ref, linw1_ref, linw2_ref, linb_ref,
              wv1_ref, wv2_ref, wv3_ref, ffnw_ref, ffnb_ref,
              lng_ref, lnb_ref, m1b_ref, agg_ref,
              w1_s, wta_s, wtb_s, w3_s, brow_s, itr_s, g_s, browm_s):
    i = pl.program_id(0)
    dot0 = lambda a, b: jax.lax.dot_general(
        a, b, (((0,), (0,)), ((), ())), preferred_element_type=F32)
    dot = lambda a, b: jax.lax.dot_general(
        a, b, (((1,), (0,)), ((), ())), preferred_element_type=F32)
    freqc = freqc_ref[...]
    tbc = tbc_ref[...]

    @pl.when(i == 0)
    def _():
        f = ffnw_ref[...]
        wv2f = dot(wv2_ref[...], f)
        w1_s[...] = dot(wv1_ref[...], f)
        wta_s[...] = dot(linw1_ref[...], wv2f)
        wtb_s[...] = dot(linw2_ref[...], wv2f)
        w3_s[...] = dot(wv3_ref[...], f)
        brow_s[...] = dot(linb_ref[...], wv2f) + ffnb_ref[...]
        agg_ref[...] = jnp.zeros_like(agg_ref)
        itc = _fcos(tbc)
        itr_s[...] = (dot0(itc, linw1_ref[...]) + dot0(itc, linw2_ref[...])
                      + linb_ref[...])
        g_s[...] = m1b_ref[...] * lng_ref[...].reshape(3 * D, 1)
        browm_s[...] = dot(lnb_ref[...], m1b_ref[...])

    t1 = t1_ref[0]
    t2 = t2_ref[0]
    t1e_t = _fcos(freqc * t1 + tbc)
    t2e_t = _fcos(freqc * t2 + tbc)
    itr = itr_s[...]
    xj = xj_ref[...]
    xi = xi_ref[...]
    user = user_ref[...]
    out = (dot(xj, w1_s[...]) + dot0(t1e_t, wta_s[...])
           + dot0(t2e_t, wtb_s[...]) + dot(user, w3_s[...]) + brow_s[...])
    q = jnp.concatenate([xi, jnp.broadcast_to(itr, xi.shape), user], axis=1)
    z = out + q
    m = jnp.mean(z, axis=-1, keepdims=True)
    zc = z - m
    var = jnp.mean(zc * zc, axis=-1, keepdims=True)
    r = lax.rsqrt(var + 1e-5)
    msgp = r * dot(zc, g_s[...]) + browm_s[...]
    seg = lax.broadcasted_iota(jnp.int32, (N2, BE), 0)
    oh = (seg == dst_ref[0]).astype(F32)
    agg_ref[...] += dot(oh, msgp)


def _tc2(t13, t23, dst3, xj, xi, user, freqc, tbc, linw1, linw2, linb,
         wv1, wv2, wv3, ffnw, ffnb, lng, lnb, m1b):
    full = lambda shape: pl.BlockSpec(shape, lambda i: tuple(0 for _ in shape))
    return pl.pallas_call(
        _tc2_body,
        grid=(NBLK,),
        in_specs=[
            pl.BlockSpec((1, 1, BE), lambda i: (i, 0, 0)),
            pl.BlockSpec((1, 1, BE), lambda i: (i, 0, 0)),
            pl.BlockSpec((1, 1, BE), lambda i: (i, 0, 0)),
            pl.BlockSpec((BE, D), lambda i: (i, 0)),
            pl.BlockSpec((BE, D), lambda i: (i, 0)),
            pl.BlockSpec((BE, D), lambda i: (i, 0)),
            full((D, 1)), full((D, 1)),
            full((D, D)), full((D, D)), full((1, D)),
            full((D, 3 * D)), full((D, 3 * D)), full((D, 3 * D)),
            full((3 * D, 3 * D)), full((1, 3 * D)),
            full((1, 3 * D)), full((1, 3 * D)), full((3 * D, D)),
        ],
        out_specs=pl.BlockSpec((N2, D), lambda i: (0, 0)),
        out_shape=jax.ShapeDtypeStruct((N2, D), F32),
        scratch_shapes=[
            pltpu.VMEM((D, 3 * D), F32),
            pltpu.VMEM((D, 3 * D), F32),
            pltpu.VMEM((D, 3 * D), F32),
            pltpu.VMEM((D, 3 * D), F32),
            pltpu.VMEM((1, 3 * D), F32),
            pltpu.VMEM((1, D), F32),
            pltpu.VMEM((3 * D, D), F32),
            pltpu.VMEM((1, D), F32),
        ],
    )(t13, t23, dst3, xj, xi, user, freqc, tbc, linw1, linw2, linb,
      wv1, wv2, wv3, ffnw, ffnb, lng, lnb, m1b)


def _tc4_body(x_ref, agg_ref, m1a_ref, m1bias_ref, m2_ref, m2b_ref, h_ref):
    i = pl.program_id(0)
    dot = lambda a, b: jax.lax.dot_general(
        a, b, (((1,), (0,)), ((), ())), preferred_element_type=F32)
    base = dot(x_ref[...], m1a_ref[...]) + m1bias_ref[...]

    def finish(acc):
        acc = jnp.where(acc > 0, acc, 0.1 * acc)
        h_ref[...] = dot(acc, m2_ref[...]) + m2b_ref[...]

    @pl.when(i < N2 // 400)
    def _():
        finish(base + agg_ref[...])

    @pl.when(i >= N2 // 400)
    def _():
        finish(base)


def _tc4(x, agg, m1a, m1bias, m2, m2b):
    BR = 400
    full = lambda shape: pl.BlockSpec(shape, lambda i: tuple(0 for _ in shape))
    return pl.pallas_call(
        _tc4_body,
        grid=(N1 // BR,),
        in_specs=[
            pl.BlockSpec((BR, D), lambda i: (i, 0)),
            pl.BlockSpec((BR, D), lambda i: (jnp.minimum(i, N2 // BR - 1), 0)),
            full((D, D)), full((1, D)),
            full((D, D)), full((1, D)),
        ],
        out_specs=pl.BlockSpec((BR, D), lambda i: (i, 0)),
        out_shape=jax.ShapeDtypeStruct((N1, D), F32),
    )(x, agg, m1a, m1bias, m2, m2b)


def _sc5_body(h_hbm, src_hbm, hs_hbm, src_v, h_b, sems):
    c = lax.axis_index("c")
    s = lax.axis_index("s")

    def run(chunk0, n):
        pltpu.sync_copy(src_hbm.at[pl.ds(chunk0 * CH, n * CH)],
                        src_v.at[pl.ds(0, n * CH)])
        descs = {}

        def fire(j):
            par = j % 2
            descs[j] = pltpu.async_copy(
                h_hbm.at[src_v.at[pl.ds(j * CH, CH)]], h_b.at[par], sems.at[par])

        fire(0)
        if n > 1:
            fire(1)
        for j in range(n):
            descs[j].wait()
            pltpu.sync_copy(h_b.at[j % 2],
                            hs_hbm.at[pl.ds((chunk0 + j) * CH, CH)])
            if j + 2 < n:
                fire(j + 2)

    @pl.when(c == 0)
    def _():
        run(s * C0, C0)

    @pl.when(c == 1)
    def _():
        run(NS * C0 + s * C1, C1)


_sc5 = pl.kernel(
    _sc5_body,
    out_type=jax.ShapeDtypeStruct((E2P, D), F32),
    mesh=_MESH,
    scratch_types=[
        pltpu.VMEM((CM * CH,), jnp.int32),
        pltpu.VMEM((2, CH, D), F32),
        pltpu.SemaphoreType.DMA((2,)),
    ],
)


def _tc6_body(hs_ref, dst_ref, h_ref, lw_ref, lb_ref, rw_ref, out_ref,
              sacc, cacc):
    i = pl.program_id(0)
    dot = lambda a, b: jax.lax.dot_general(
        a, b, (((1,), (0,)), ((), ())), preferred_element_type=F32)
    seg = lax.broadcasted_iota(jnp.int32, (N2, BE), 0)
    oh = (seg == dst_ref[0]).astype(F32)

    @pl.when(i == 0)
    def _():
        sacc[...] = jnp.zeros_like(sacc)
        cacc[...] = jnp.zeros_like(cacc)

    sacc[...] += dot(oh, hs_ref[...])
    cacc[...] += dot(oh, jnp.ones((BE, 8), F32))

    @pl.when(i == NBLK - 1)
    def _():
        cnt = cacc[:, 0:1]
        mean = sacc[...] / jnp.maximum(cnt, 1.0)
        out = dot(mean, lw_ref[...]) + lb_ref[...] + dot(h_ref[...], rw_ref[...])
        nrm = jnp.sqrt(jnp.sum(out * out, axis=-1, keepdims=True))
        out_ref[...] = out / jnp.maximum(nrm, 1e-12)


def _tc6(hs, dst3, h, lw, lb, rw):
    full = lambda shape: pl.BlockSpec(shape, lambda i: tuple(0 for _ in shape))
    return pl.pallas_call(
        _tc6_body,
        grid=(NBLK,),
        in_specs=[
            pl.BlockSpec((BE, D), lambda i: (i, 0)),
            pl.BlockSpec((1, 1, BE), lambda i: (i, 0, 0)),
            pl.BlockSpec((N2, D), lambda i: (0, 0)),
            full((D, D)), full((1, D)), full((D, D)),
        ],
        out_specs=pl.BlockSpec((N2, D), lambda i: (0, 0)),
        out_shape=jax.ShapeDtypeStruct((N2, D), F32),
        scratch_shapes=[
            pltpu.VMEM((N2, D), F32),
            pltpu.VMEM((N2, 8), F32),
        ],
    )(hs, dst3, h, lw, lb, rw)


def kernel(x, edge_index1, e_id1, edge_index2, e_id2, emb, tg_edge_attr,
           params, size1, size2):
    p = params
    uidx_tab = tg_edge_attr[:, 2].astype(jnp.int32)
    t1_tab = tg_edge_attr[:, 0]
    t2_tab = tg_edge_attr[:, 1]
    pad = E2P - E2
    eid3 = jnp.pad(e_id2.astype(jnp.int32), (0, pad))
    src3 = jnp.pad(edge_index2[0].astype(jnp.int32), (0, pad))
    dst_pad = jnp.pad(edge_index2[1].astype(jnp.int32), (0, pad),
                      constant_values=N2)
    dst3 = dst_pad
    dstl = dst_pad.reshape(NBLK, 1, BE)

    row = lambda v: v.reshape(1, -1)
    col = lambda v: v.reshape(-1, 1)
    t1g, t2g, xj, xi, user = _sc1(t1_tab, t2_tab, uidx_tab,
                                  eid3, src3, dst3, x, emb)
    t13 = t1g.reshape(NBLK, 1, BE)
    t23 = t2g.reshape(NBLK, 1, BE)
    inv2pi = F32(1.0 / (2.0 * jnp.pi))
    agg = _tc2(t13, t23, dstl, xj, xi, user,
               col(p['basic_freq']) * inv2pi, col(p['t_bias']) * inv2pi,
               p['lin_w'][:D], p['lin_w'][D:], row(p['lin_b']),
               p['wv'][:D], p['wv'][D:2 * D], p['wv'][2 * D:],
               p['ffn_w'], row(p['ffn_b']),
               row(p['ln_g']), row(p['ln_b']), p['m1_w'][D:])
    h = _tc4(x, agg, p['m1_w'][:D], row(p['m1_b']),
             p['m2_w'], row(p['m2_b']))
    hs = _sc5(h, src3)
    return _tc6(hs, dstl, h, p['sage1_lw'], row(p['sage1_lb']), p['sage1_rw'])


# speedup vs baseline: 1.1253x; 1.0880x over previous
import jax
import jax.numpy as jnp
from jax import lax
from jax.experimental import pallas as pl
from jax.experimental.pallas import tpu as pltpu
from jax.experimental.pallas import tpu_sc as plsc

D = 128
N0, N1, N2 = 50000, 10000, 2000
ETOT = 800000
E2 = 20000
NC, NS, L = 2, 16, 16
NW = NC * NS
CH = 128
K = 5
EPW = CH * K
E2P = NW * EPW
BE = 1024
NBLK = E2P // BE
F32 = jnp.float32

_MESH = plsc.VectorSubcoreMesh(core_axis_name="c", subcore_axis_name="s")


C0 = 5
C1 = 5
NCHK = NW * K
CM = max(C0, C1)


def _sc1_body(t1t_hbm, t2t_hbm, uidxt_hbm, eid_hbm, src_hbm, dst_hbm,
              x_hbm, emb_hbm,
              t1g_hbm, t2g_hbm, xj_hbm, xi_hbm, user_hbm,
              eid_v, src_v, dst_v, t1_b, t2_b, uidx_b,
              xj_b, xi_b, user_b, sems):
    c = lax.axis_index("c")
    s = lax.axis_index("s")

    def run(chunk0, n):
        pltpu.sync_copy(eid_hbm.at[pl.ds(chunk0 * CH, n * CH)],
                        eid_v.at[pl.ds(0, n * CH)])
        pltpu.sync_copy(src_hbm.at[pl.ds(chunk0 * CH, n * CH)],
                        src_v.at[pl.ds(0, n * CH)])
        pltpu.sync_copy(dst_hbm.at[pl.ds(chunk0 * CH, n * CH)],
                        dst_v.at[pl.ds(0, n * CH)])
        descs = {}

        def fire_indep(j):
            par = j % 2
            descs[(j, 0)] = pltpu.async_copy(
                t1t_hbm.at[eid_v.at[pl.ds(j * CH, CH)]], t1_b.at[par], sems.at[par, 0])
            descs[(j, 1)] = pltpu.async_copy(
                t2t_hbm.at[eid_v.at[pl.ds(j * CH, CH)]], t2_b.at[par], sems.at[par, 1])
            descs[(j, 2)] = pltpu.async_copy(
                uidxt_hbm.at[eid_v.at[pl.ds(j * CH, CH)]], uidx_b.at[par], sems.at[par, 2])
            descs[(j, 3)] = pltpu.async_copy(
                x_hbm.at[src_v.at[pl.ds(j * CH, CH)]], xj_b.at[par], sems.at[par, 3])
            descs[(j, 4)] = pltpu.async_copy(
                x_hbm.at[dst_v.at[pl.ds(j * CH, CH)]], xi_b.at[par], sems.at[par, 4])

        def fire_emb(j):
            par = j % 2
            descs[(j, 2)].wait()
            descs[(j, 5)] = pltpu.async_copy(
                emb_hbm.at[uidx_b.at[par]], user_b.at[par], sems.at[par, 5])

        def drain_store(j):
            par = j % 2
            for t in (0, 1, 3, 4, 5):
                descs[(j, t)].wait()
            base = (chunk0 + j) * CH
            pltpu.sync_copy(t1_b.at[par], t1g_hbm.at[pl.ds(base, CH)])
            pltpu.sync_copy(t2_b.at[par], t2g_hbm.at[pl.ds(base, CH)])
            pltpu.sync_copy(xj_b.at[par], xj_hbm.at[pl.ds(base, CH)])
            pltpu.sync_copy(xi_b.at[par], xi_hbm.at[pl.ds(base, CH)])
            pltpu.sync_copy(user_b.at[par], user_hbm.at[pl.ds(base, CH)])

        fire_indep(0)
        fire_emb(0)
        if n > 1:
            fire_indep(1)
        for j in range(n):
            drain_store(j)
            if j + 1 < n:
                fire_emb(j + 1)
            if j + 2 < n:
                fire_indep(j + 2)

    @pl.when(c == 0)
    def _():
        run(s * C0, C0)

    @pl.when(c == 1)
    def _():
        run(NS * C0 + s * C1, C1)


_sc1 = pl.kernel(
    _sc1_body,
    out_type=[
        jax.ShapeDtypeStruct((E2P,), F32),
        jax.ShapeDtypeStruct((E2P,), F32),
        jax.ShapeDtypeStruct((E2P, D), F32),
        jax.ShapeDtypeStruct((E2P, D), F32),
        jax.ShapeDtypeStruct((E2P, D), F32),
    ],
    mesh=_MESH,
    scratch_types=[
        pltpu.VMEM((CM * CH,), jnp.int32),
        pltpu.VMEM((CM * CH,), jnp.int32),
        pltpu.VMEM((CM * CH,), jnp.int32),
        pltpu.VMEM((2, CH), F32),
        pltpu.VMEM((2, CH), F32),
        pltpu.VMEM((2, CH), jnp.int32),
        pltpu.VMEM((2, CH, D), F32),
        pltpu.VMEM((2, CH, D), F32),
        pltpu.VMEM((2, CH, D), F32),
        pltpu.SemaphoreType.DMA((2, 6)),
    ],
)


_CC = (0.9999982503105564, -19.738913224823705, 64.92748557653424,
       -85.26424585397747, 58.77468699833364, -21.06805280070973)


def _fcos(y):
    fr = y - jnp.round(y)
    v = fr * fr
    acc = jnp.full_like(v, _CC[5])
    for k in (4, 3, 2, 1, 0):
        acc = acc * v + _CC[k]
    return acc


def _tc2_body(t1_ref, t2_ref, dst_ref, xj_ref, xi_ref, user_ref,
              freqc_ref, tbc_ref, linw1_ref, linw2_ref, linb_ref,
              wv1_ref, wv2_ref, wv3_ref, ffnw_ref, ffnb_ref,
              lng_ref, lnb_ref, m1b_ref, agg_ref,
              w1_s, wta_s, wtb_s, w3_s, brow_s, itr_s, g_s, browm_s):
    i = pl.program_id(0)
    dot0 = lambda a, b: jax.lax.dot_general(
        a, b, (((0,), (0,)), ((), ())), preferred_element_type=F32)
    dot = lambda a, b: jax.lax.dot_general(
        a, b, (((1,), (0,)), ((), ())), preferred_element_type=F32)
    freqc = freqc_ref[...]
    tbc = tbc_ref[...]

    @pl.when(i == 0)
    def _():
        f = ffnw_ref[...]
        wv2f = dot(wv2_ref[...], f)
        w1_s[...] = dot(wv1_ref[...], f)
        wta_s[...] = dot(linw1_ref[...], wv2f)
        wtb_s[...] = dot(linw2_ref[...], wv2f)
        w3_s[...] = dot(wv3_ref[...], f)
        brow_s[...] = dot(linb_ref[...], wv2f) + ffnb_ref[...]
        agg_ref[...] = jnp.zeros_like(agg_ref)
        itc = _fcos(tbc)
        itr_s[...] = (dot0(itc, linw1_ref[...]) + dot0(itc, linw2_ref[...])
                      + linb_ref[...])
        g_s[...] = m1b_ref[...] * lng_ref[...].reshape(3 * D, 1)
        browm_s[...] = dot(lnb_ref[...], m1b_ref[...])

    t1 = t1_ref[0]
    t2 = t2_ref[0]
    t1e_t = _fcos(freqc * t1 + tbc)
    t2e_t = _fcos(freqc * t2 + tbc)
    itr = itr_s[...]
    xj = xj_ref[...]
    xi = xi_ref[...]
    user = user_ref[...]
    out = (dot(xj, w1_s[...]) + dot0(t1e_t, wta_s[...])
           + dot0(t2e_t, wtb_s[...]) + dot(user, w3_s[...]) + brow_s[...])
    q = jnp.concatenate([xi, jnp.broadcast_to(itr, xi.shape), user], axis=1)
    z = out + q
    m = jnp.mean(z, axis=-1, keepdims=True)
    zc = z - m
    var = jnp.mean(zc * zc, axis=-1, keepdims=True)
    r = lax.rsqrt(var + 1e-5)
    msgp = r * dot(zc, g_s[...]) + browm_s[...]
    seg = lax.broadcasted_iota(jnp.int32, (N2, BE), 0)
    oh = (seg == dst_ref[0]).astype(F32)
    agg_ref[...] += dot(oh, msgp)


def _tc2(t13, t23, dst3, xj, xi, user, freqc, tbc, linw1, linw2, linb,
         wv1, wv2, wv3, ffnw, ffnb, lng, lnb, m1b):
    full = lambda shape: pl.BlockSpec(shape, lambda i: tuple(0 for _ in shape))
    return pl.pallas_call(
        _tc2_body,
        grid=(NBLK,),
        in_specs=[
            pl.BlockSpec((1, 1, BE), lambda i: (i, 0, 0)),
            pl.BlockSpec((1, 1, BE), lambda i: (i, 0, 0)),
            pl.BlockSpec((1, 1, BE), lambda i: (i, 0, 0)),
            pl.BlockSpec((BE, D), lambda i: (i, 0)),
            pl.BlockSpec((BE, D), lambda i: (i, 0)),
            pl.BlockSpec((BE, D), lambda i: (i, 0)),
            full((D, 1)), full((D, 1)),
            full((D, D)), full((D, D)), full((1, D)),
            full((D, 3 * D)), full((D, 3 * D)), full((D, 3 * D)),
            full((3 * D, 3 * D)), full((1, 3 * D)),
            full((1, 3 * D)), full((1, 3 * D)), full((3 * D, D)),
        ],
        out_specs=pl.BlockSpec((N2, D), lambda i: (0, 0)),
        out_shape=jax.ShapeDtypeStruct((N2, D), F32),
        scratch_shapes=[
            pltpu.VMEM((D, 3 * D), F32),
            pltpu.VMEM((D, 3 * D), F32),
            pltpu.VMEM((D, 3 * D), F32),
            pltpu.VMEM((D, 3 * D), F32),
            pltpu.VMEM((1, 3 * D), F32),
            pltpu.VMEM((1, D), F32),
            pltpu.VMEM((3 * D, D), F32),
            pltpu.VMEM((1, D), F32),
        ],
    )(t13, t23, dst3, xj, xi, user, freqc, tbc, linw1, linw2, linb,
      wv1, wv2, wv3, ffnw, ffnb, lng, lnb, m1b)


def _tc4_body(x_ref, agg_ref, m1a_ref, m1bias_ref, m2_ref, m2b_ref, h_ref):
    i = pl.program_id(0)
    dot = lambda a, b: jax.lax.dot_general(
        a, b, (((1,), (0,)), ((), ())), preferred_element_type=F32)
    base = dot(x_ref[...], m1a_ref[...]) + m1bias_ref[...]

    def finish(acc):
        acc = jnp.where(acc > 0, acc, 0.1 * acc)
        h_ref[...] = dot(acc, m2_ref[...]) + m2b_ref[...]

    @pl.when(i < N2 // 400)
    def _():
        finish(base + agg_ref[...])

    @pl.when(i >= N2 // 400)
    def _():
        finish(base)


def _tc4(x, agg, m1a, m1bias, m2, m2b):
    BR = 400
    full = lambda shape: pl.BlockSpec(shape, lambda i: tuple(0 for _ in shape))
    return pl.pallas_call(
        _tc4_body,
        grid=(N1 // BR,),
        in_specs=[
            pl.BlockSpec((BR, D), lambda i: (i, 0)),
            pl.BlockSpec((BR, D), lambda i: (jnp.minimum(i, N2 // BR - 1), 0)),
            full((D, D)), full((1, D)),
            full((D, D)), full((1, D)),
        ],
        out_specs=pl.BlockSpec((BR, D), lambda i: (i, 0)),
        out_shape=jax.ShapeDtypeStruct((N1, D), F32),
    )(x, agg, m1a, m1bias, m2, m2b)


def _sc5_body(h_hbm, src_hbm, hs_hbm, src_v, h_b, sems):
    c = lax.axis_index("c")
    s = lax.axis_index("s")

    def run(chunk0, n):
        pltpu.sync_copy(src_hbm.at[pl.ds(chunk0 * CH, n * CH)],
                        src_v.at[pl.ds(0, n * CH)])
        descs = {}

        def fire(j):
            par = j % 2
            descs[j] = pltpu.async_copy(
                h_hbm.at[src_v.at[pl.ds(j * CH, CH)]], h_b.at[par], sems.at[par])

        fire(0)
        if n > 1:
            fire(1)
        for j in range(n):
            descs[j].wait()
            pltpu.sync_copy(h_b.at[j % 2],
                            hs_hbm.at[pl.ds((chunk0 + j) * CH, CH)])
            if j + 2 < n:
                fire(j + 2)

    @pl.when(c == 0)
    def _():
        run(s * C0, C0)

    @pl.when(c == 1)
    def _():
        run(NS * C0 + s * C1, C1)


_sc5 = pl.kernel(
    _sc5_body,
    out_type=jax.ShapeDtypeStruct((E2P, D), F32),
    mesh=_MESH,
    scratch_types=[
        pltpu.VMEM((CM * CH,), jnp.int32),
        pltpu.VMEM((2, CH, D), F32),
        pltpu.SemaphoreType.DMA((2,)),
    ],
)


def _tc6_body(hs_ref, dst_ref, h_ref, lw_ref, lb_ref, rw_ref, out_ref,
              sacc, cacc):
    i = pl.program_id(0)
    dot = lambda a, b: jax.lax.dot_general(
        a, b, (((1,), (0,)), ((), ())), preferred_element_type=F32)
    seg = lax.broadcasted_iota(jnp.int32, (N2, BE), 0)
    oh = (seg == dst_ref[0]).astype(F32)

    @pl.when(i == 0)
    def _():
        sacc[...] = jnp.zeros_like(sacc)
        cacc[...] = jnp.zeros_like(cacc)

    sacc[...] += dot(oh, hs_ref[...])
    cacc[...] += dot(oh, jnp.ones((BE, 8), F32))

    @pl.when(i == NBLK - 1)
    def _():
        cnt = cacc[:, 0:1]
        mean = sacc[...] / jnp.maximum(cnt, 1.0)
        out = dot(mean, lw_ref[...]) + lb_ref[...] + dot(h_ref[...], rw_ref[...])
        nrm = jnp.sqrt(jnp.sum(out * out, axis=-1, keepdims=True))
        out_ref[...] = out / jnp.maximum(nrm, 1e-12)


def _tc6(hs, dst3, h, lw, lb, rw):
    full = lambda shape: pl.BlockSpec(shape, lambda i: tuple(0 for _ in shape))
    return pl.pallas_call(
        _tc6_body,
        grid=(NBLK,),
        in_specs=[
            pl.BlockSpec((BE, D), lambda i: (i, 0)),
            pl.BlockSpec((1, 1, BE), lambda i: (i, 0, 0)),
            pl.BlockSpec((N2, D), lambda i: (0, 0)),
            full((D, D)), full((1, D)), full((D, D)),
        ],
        out_specs=pl.BlockSpec((N2, D), lambda i: (0, 0)),
        out_shape=jax.ShapeDtypeStruct((N2, D), F32),
        scratch_shapes=[
            pltpu.VMEM((N2, D), F32),
            pltpu.VMEM((N2, 8), F32),
        ],
    )(hs, dst3, h, lw, lb, rw)


def kernel(x, edge_index1, e_id1, edge_index2, e_id2, emb, tg_edge_attr,
           params, size1, size2):
    p = params
    uidx_tab = tg_edge_attr[:, 2].astype(jnp.int32)
    t1_tab = tg_edge_attr[:, 0]
    t2_tab = tg_edge_attr[:, 1]
    pad = E2P - E2
    eid3 = jnp.pad(e_id2.astype(jnp.int32), (0, pad))
    src3 = jnp.pad(edge_index2[0].astype(jnp.int32), (0, pad))
    dst_pad = jnp.pad(edge_index2[1].astype(jnp.int32), (0, pad),
                      constant_values=N2)
    dst3 = dst_pad
    dstl = dst_pad.reshape(NBLK, 1, BE)

    row = lambda v: v.reshape(1, -1)
    col = lambda v: v.reshape(-1, 1)
    t1g, t2g, xj, xi, user = _sc1(t1_tab, t2_tab, uidx_tab,
                                  eid3, src3, dst3, x, emb)
    t13 = t1g.reshape(NBLK, 1, BE)
    t23 = t2g.reshape(NBLK, 1, BE)
    inv2pi = F32(1.0 / (2.0 * jnp.pi))
    agg = _tc2(t13, t23, dstl, xj, xi, user,
               col(p['basic_freq']) * inv2pi, col(p['t_bias']) * inv2pi,
               p['lin_w'][:D], p['lin_w'][D:], row(p['lin_b']),
               p['wv'][:D], p['wv'][D:2 * D], p['wv'][2 * D:],
               p['ffn_w'], row(p['ffn_b']),
               row(p['ln_g']), row(p['ln_b']), p['m1_w'][D:])
    h = _tc4(x, agg, p['m1_w'][:D], row(p['m1_b']),
             p['m2_w'], row(p['m2_b']))
    hs = _sc5(h, src3)
    return _tc6(hs, dstl, h, p['sage1_lw'], row(p['sage1_lb']), p['sage1_rw'])
